# baseline (speedup 1.0000x reference)
import functools

import jax
import jax.numpy as jnp
from jax import lax
from jax.experimental import pallas as pl
from jax.experimental.pallas import tpu as pltpu
from jax.experimental.pallas import tpu_sc as plsc

V = 65536
D = 256
B = 49152
NC, NS, L = 2, 16, 16
NW = NC * NS
CHUNK = V // NW
NVR = CHUNK // L
BPW = B // NW
GB = 128
NGB = BPW // GB
IDX_ROWS = B // GB + 1

_mesh = plsc.VectorSubcoreMesh(core_axis_name="c", subcore_axis_name="s")
_params = pltpu.CompilerParams(needs_layout_passes=False)


@functools.partial(
    pl.kernel,
    out_type=jax.ShapeDtypeStruct((IDX_ROWS * GB,), jnp.int32),
    mesh=_mesh,
    scratch_types=[
        pltpu.VMEM((V,), jnp.int32),
        pltpu.VMEM((CHUNK,), jnp.int32),
        pltpu.VMEM((NVR // 8, GB), jnp.int32),
        pltpu.SemaphoreType.DMA,
    ],
    compiler_params=_params,
)
def _compact(mask_hbm, idx_hbm, maskbuf, posbuf, rankbuf, sem):
    wid = lax.axis_index("s") * NC + lax.axis_index("c")
    pltpu.sync_copy(mask_hbm, maskbuf)

    def body8(c, acc):
        for k in range(8):
            acc = acc + plsc.all_reduce_population_count(
                maskbuf[pl.ds(c * (8 * L) + k * L, L)] > 0)
        return acc

    runpre = lax.fori_loop(0, wid * (NVR // 8), body8,
                           jnp.zeros((L,), jnp.int32))

    lane = lax.iota(jnp.int32, L)
    base = wid * CHUNK
    for i in range(NVR):
        m = maskbuf[pl.ds(base + i * L, L)]
        ison = m > 0
        pos = jnp.full((L,), base + i * L, jnp.int32) + lane
        r, c0 = i // 8, (i % 8) * L
        plsc.store_compressed(posbuf.at[pl.ds(i * L, L)], pos, mask=ison)
        cnt = plsc.all_reduce_population_count(ison)
        rankbuf[r, pl.ds(c0, L)] = jnp.where(lane < cnt, runpre + lane,
                                             B + lane)
        runpre = runpre + cnt

    copies = [
        pltpu.async_copy(posbuf.at[pl.ds(j * GB, GB)],
                         idx_hbm.at[rankbuf.at[j]], sem)
        for j in range(NVR // 8)
    ]
    for cpy in copies:
        cpy.wait()


@functools.partial(
    pl.kernel,
    out_type=jax.ShapeDtypeStruct((B, D), jnp.float32),
    mesh=_mesh,
    scratch_types=[
        pltpu.VMEM((NGB, GB), jnp.int32),
        pltpu.VMEM((2, GB, D), jnp.float32),
        pltpu.SemaphoreType.DMA,
        pltpu.SemaphoreType.DMA,
    ],
    compiler_params=_params,
)
def _gather(states_hbm, idx_hbm, out_hbm, idxbuf, rowbuf, sem0, sem1):
    wid = lax.axis_index("s") * NC + lax.axis_index("c")
    loads = [
        pltpu.async_copy(idx_hbm.at[pl.ds(wid * BPW + j * GB, GB)],
                         idxbuf.at[j], sem0)
        for j in range(NGB)
    ]
    for ld in loads:
        ld.wait()
    sems = (sem0, sem1)
    obase = wid * BPW
    h = pltpu.async_copy(states_hbm.at[idxbuf.at[0]], rowbuf.at[0], sems[0])
    for j in range(NGB):
        if j + 1 < NGB:
            h_next = pltpu.async_copy(
                states_hbm.at[idxbuf.at[j + 1]], rowbuf.at[(j + 1) % 2],
                sems[(j + 1) % 2])
        h.wait()
        pltpu.sync_copy(rowbuf.at[j % 2], out_hbm.at[pl.ds(obase + j * GB, GB)])
        if j + 1 < NGB:
            h = h_next


def kernel(inputs, states, importance_scores, active_mask):
    idx_flat = _compact(active_mask.astype(jnp.int32))
    return _gather(states, idx_flat)


# speedup vs baseline: 9.7007x; 9.7007x over previous
import functools

import jax
import jax.numpy as jnp
from jax import lax
from jax.experimental import pallas as pl
from jax.experimental.pallas import tpu as pltpu
from jax.experimental.pallas import tpu_sc as plsc

V = 65536
D = 256
B = 49152
NC, NS, L = 2, 16, 16
NW = NC * NS
CHUNK = V // NW
NVR = CHUNK // L
BPW = B // NW
GB = 128
NGB = BPW // GB
IDX_SIZE = B + V

_mesh = plsc.VectorSubcoreMesh(core_axis_name="c", subcore_axis_name="s")
_params = pltpu.CompilerParams(needs_layout_passes=False)


@functools.partial(
    pl.kernel,
    out_type=jax.ShapeDtypeStruct((IDX_SIZE,), jnp.int32),
    mesh=_mesh,
    scratch_types=[
        pltpu.VMEM((V,), jnp.int32),
        pltpu.VMEM((CHUNK,), jnp.int32),
        pltpu.VMEM((NVR // 8, GB), jnp.int32),
        pltpu.SemaphoreType.DMA,
    ],
    compiler_params=_params,
)
def _compact(mask_hbm, idx_hbm, maskbuf, posbuf, rankbuf, sem):
    wid = lax.axis_index("s") * NC + lax.axis_index("c")
    pltpu.sync_copy(mask_hbm, maskbuf)

    def body8(c, acc):
        for k in range(8):
            acc = acc + plsc.all_reduce_population_count(
                maskbuf[pl.ds(c * (8 * L) + k * L, L)] > 0)
        return acc

    runpre = lax.fori_loop(0, wid * (NVR // 8), body8,
                           jnp.zeros((L,), jnp.int32))

    lane = lax.iota(jnp.int32, L)
    base = wid * CHUNK
    for i in range(NVR):
        m = maskbuf[pl.ds(base + i * L, L)]
        ison = m > 0
        pos = jnp.full((L,), base + i * L, jnp.int32) + lane
        r, c0 = i // 8, (i % 8) * L
        plsc.store_compressed(posbuf.at[pl.ds(i * L, L)], pos, mask=ison)
        cnt = plsc.all_reduce_population_count(ison)
        rankbuf[r, pl.ds(c0, L)] = jnp.where(lane < cnt, runpre + lane,
                                             B + pos)
        runpre = runpre + cnt

    copies = [
        pltpu.async_copy(posbuf.at[pl.ds(j * GB, GB)],
                         idx_hbm.at[rankbuf.at[j]], sem)
        for j in range(NVR // 8)
    ]
    for cpy in copies:
        cpy.wait()


@functools.partial(
    pl.kernel,
    out_type=jax.ShapeDtypeStruct((B, D), jnp.float32),
    mesh=_mesh,
    scratch_types=[
        pltpu.VMEM((NGB, GB), jnp.int32),
        pltpu.VMEM((2, GB, D), jnp.float32),
        pltpu.SemaphoreType.DMA,
        pltpu.SemaphoreType.DMA,
    ],
    compiler_params=_params,
)
def _gather(states_hbm, idx_hbm, out_hbm, idxbuf, rowbuf, sem0, sem1):
    wid = lax.axis_index("s") * NC + lax.axis_index("c")
    loads = [
        pltpu.async_copy(idx_hbm.at[pl.ds(wid * BPW + j * GB, GB)],
                         idxbuf.at[j], sem0)
        for j in range(NGB)
    ]
    for ld in loads:
        ld.wait()
    sems = (sem0, sem1)
    obase = wid * BPW
    h = pltpu.async_copy(states_hbm.at[idxbuf.at[0]], rowbuf.at[0], sems[0])
    for j in range(NGB):
        if j + 1 < NGB:
            h_next = pltpu.async_copy(
                states_hbm.at[idxbuf.at[j + 1]], rowbuf.at[(j + 1) % 2],
                sems[(j + 1) % 2])
        h.wait()
        pltpu.sync_copy(rowbuf.at[j % 2], out_hbm.at[pl.ds(obase + j * GB, GB)])
        if j + 1 < NGB:
            h = h_next


def kernel(inputs, states, importance_scores, active_mask):
    idx_flat = _compact(active_mask.astype(jnp.int32))
    return _gather(states, idx_flat)


# speedup vs baseline: 9.7148x; 1.0015x over previous
import functools

import jax
import jax.numpy as jnp
from jax import lax
from jax.experimental import pallas as pl
from jax.experimental.pallas import tpu as pltpu
from jax.experimental.pallas import tpu_sc as plsc

V = 65536
D = 256
B = 49152
NC, NS, L = 2, 16, 16
NW = NC * NS
CHUNK = V // NW
NVR = CHUNK // L
BPW = B // NW
GB = 128
NGB = BPW // GB
IDX_SIZE = B + V

_mesh = plsc.VectorSubcoreMesh(core_axis_name="c", subcore_axis_name="s")
_params = pltpu.CompilerParams(needs_layout_passes=False)


@functools.partial(
    pl.kernel,
    out_type=jax.ShapeDtypeStruct((IDX_SIZE,), jnp.int32),
    mesh=_mesh,
    scratch_types=[
        pltpu.VMEM((V,), jnp.int32),
        pltpu.VMEM((CHUNK,), jnp.int32),
        pltpu.VMEM((NVR // 8, GB), jnp.int32),
        pltpu.SemaphoreType.DMA,
    ],
    compiler_params=_params,
)
def _compact(mask_hbm, idx_hbm, maskbuf, posbuf, rankbuf, sem):
    wid = lax.axis_index("s") * NC + lax.axis_index("c")
    with jax.named_scope("mask_load"):
        pltpu.sync_copy(mask_hbm, maskbuf)

    def body8(c, acc):
        for k in range(8):
            acc = acc + plsc.all_reduce_population_count(
                maskbuf[pl.ds(c * (8 * L) + k * L, L)] > 0)
        return acc

    with jax.named_scope("prefix"):
        runpre = lax.fori_loop(0, wid * (NVR // 8), body8,
                               jnp.zeros((L,), jnp.int32))

    lane = lax.iota(jnp.int32, L)
    base = wid * CHUNK
    with jax.named_scope("compress"):
        for i in range(NVR):
            m = maskbuf[pl.ds(base + i * L, L)]
            ison = m > 0
            pos = jnp.full((L,), base + i * L, jnp.int32) + lane
            r, c0 = i // 8, (i % 8) * L
            plsc.store_compressed(posbuf.at[pl.ds(i * L, L)], pos, mask=ison)
            cnt = plsc.all_reduce_population_count(ison)
            rankbuf[r, pl.ds(c0, L)] = jnp.where(lane < cnt, runpre + lane,
                                                 B + pos)
            runpre = runpre + cnt

    with jax.named_scope("scatter"):
        copies = [
            pltpu.async_copy(posbuf.at[pl.ds(j * GB, GB)],
                             idx_hbm.at[rankbuf.at[j]], sem)
            for j in range(NVR // 8)
        ]
        for cpy in copies:
            cpy.wait()


@functools.partial(
    pl.kernel,
    out_type=jax.ShapeDtypeStruct((B, D), jnp.float32),
    mesh=_mesh,
    scratch_types=[
        pltpu.VMEM((NGB, GB), jnp.int32),
        pltpu.VMEM((2, GB, D), jnp.float32),
        pltpu.SemaphoreType.DMA,
        pltpu.SemaphoreType.DMA,
    ],
    compiler_params=_params,
)
def _gather(states_hbm, idx_hbm, out_hbm, idxbuf, rowbuf, sem0, sem1):
    wid = lax.axis_index("s") * NC + lax.axis_index("c")
    loads = [
        pltpu.async_copy(idx_hbm.at[pl.ds(wid * BPW + j * GB, GB)],
                         idxbuf.at[j], sem0)
        for j in range(NGB)
    ]
    for ld in loads:
        ld.wait()
    sems = (sem0, sem1)
    obase = wid * BPW
    h = pltpu.async_copy(states_hbm.at[idxbuf.at[0]], rowbuf.at[0], sems[0])
    for j in range(NGB):
        if j + 1 < NGB:
            h_next = pltpu.async_copy(
                states_hbm.at[idxbuf.at[j + 1]], rowbuf.at[(j + 1) % 2],
                sems[(j + 1) % 2])
        h.wait()
        pltpu.sync_copy(rowbuf.at[j % 2], out_hbm.at[pl.ds(obase + j * GB, GB)])
        if j + 1 < NGB:
            h = h_next


def kernel(inputs, states, importance_scores, active_mask):
    idx_flat = _compact(active_mask.astype(jnp.int32))
    return _gather(states, idx_flat)


# speedup vs baseline: 37.5422x; 3.8644x over previous
import functools

import jax
import jax.numpy as jnp
from jax import lax
from jax.experimental import pallas as pl
from jax.experimental.pallas import tpu as pltpu
from jax.experimental.pallas import tpu_sc as plsc

V = 65536
D = 256
B = 49152
NC, NS, L = 2, 16, 16
NW = NC * NS
NV = V // L
BPW = B // NW
GB = 96
NGB = BPW // GB
IBUF = BPW + L

_mesh = plsc.VectorSubcoreMesh(core_axis_name="c", subcore_axis_name="s")
_params = pltpu.CompilerParams(needs_layout_passes=False)


@functools.partial(
    pl.kernel,
    out_type=jax.ShapeDtypeStruct((B, D), jnp.float32),
    mesh=_mesh,
    scratch_types=[
        pltpu.VMEM((V,), jnp.int32),
        pltpu.VMEM((IBUF,), jnp.int32),
        pltpu.VMEM((2, GB, D), jnp.float32),
        pltpu.SemaphoreType.DMA,
        pltpu.SemaphoreType.DMA,
    ],
    compiler_params=_params,
)
def _compact_gather(mask_hbm, states_hbm, out_hbm, maskbuf, ibuf, rowbuf,
                    sem0, sem1):
    wid = lax.axis_index("s") * NC + lax.axis_index("c")
    with jax.named_scope("mask_load"):
        pltpu.sync_copy(mask_hbm, maskbuf)

    lane = lax.iota(jnp.int32, L)
    target = jnp.full((L,), wid * BPW, jnp.int32)

    def lbody(c, carry):
        acc, startv, accsel = carry
        for k in range(8):
            cnt = plsc.all_reduce_population_count(
                maskbuf[pl.ds(c * (8 * L) + k * L, L)] > 0)
            take = (acc + cnt) <= target
            startv = startv + jnp.where(take, 1, 0)
            accsel = jnp.where(take, acc + cnt, accsel)
            acc = acc + cnt
        return acc, startv, accsel

    with jax.named_scope("locate"):
        zero = jnp.zeros((L,), jnp.int32)
        _, startv, accsel = lax.fori_loop(0, NV // 8, lbody,
                                          (zero, zero, zero))
        k0 = target - accsel
        sv = jnp.max(startv)

    with jax.named_scope("walk"):
        m0 = maskbuf[pl.ds(sv * L, L)]
        ison0 = m0 > 0
        pref0 = plsc.cumsum(jnp.where(ison0, 1, 0)) - jnp.where(ison0, 1, 0)
        keep0 = ison0 & (pref0 >= k0)
        pos0 = jnp.full((L,), sv * L, jnp.int32) + lane
        plsc.store_compressed(ibuf.at[pl.ds(0, L)], pos0, mask=keep0)
        coll0 = jnp.sum(jnp.where(keep0, 1, 0))

        def wcond(carry):
            coll, vi = carry
            return (coll < BPW) & (vi < NV)

        def wbody(carry):
            coll, vi = carry
            m = maskbuf[pl.ds(vi * L, L)]
            ison = m > 0
            pos = jnp.full((L,), vi * L, jnp.int32) + lane
            plsc.store_compressed(ibuf.at[pl.ds(coll, L)], pos, mask=ison)
            return coll + jnp.sum(jnp.where(ison, 1, 0)), vi + 1

        lax.while_loop(wcond, wbody, (coll0, sv + 1))

    with jax.named_scope("gather"):
        obase = wid * BPW
        h = pltpu.async_copy(states_hbm.at[ibuf.at[pl.ds(0, GB)]],
                             rowbuf.at[0], sem0)
        sems = (sem0, sem1)
        for j in range(NGB):
            if j + 1 < NGB:
                h_next = pltpu.async_copy(
                    states_hbm.at[ibuf.at[pl.ds((j + 1) * GB, GB)]],
                    rowbuf.at[(j + 1) % 2], sems[(j + 1) % 2])
            h.wait()
            pltpu.sync_copy(rowbuf.at[j % 2],
                            out_hbm.at[pl.ds(obase + j * GB, GB)])
            if j + 1 < NGB:
                h = h_next


def kernel(inputs, states, importance_scores, active_mask):
    return _compact_gather(active_mask.astype(jnp.int32), states)


# speedup vs baseline: 43.5244x; 1.1593x over previous
import functools

import jax
import jax.numpy as jnp
from jax import lax
from jax.experimental import pallas as pl
from jax.experimental.pallas import tpu as pltpu
from jax.experimental.pallas import tpu_sc as plsc

V = 65536
D = 256
B = 49152
NC, NS, L = 2, 16, 16
NW = NC * NS
BPW = B // NW
SEG = V // NS
SVR = SEG // L
WWIN = 18432
NWV = WWIN // L
GB = 96
NGB = BPW // GB
NBUF = 4
AHEAD = 2
IBUF = BPW + L

_mesh = plsc.VectorSubcoreMesh(core_axis_name="c", subcore_axis_name="s")
_params = pltpu.CompilerParams(needs_layout_passes=False)


@functools.partial(
    pl.kernel,
    out_type=jax.ShapeDtypeStruct((B, D), jnp.float32),
    mesh=_mesh,
    scratch_types=[
        pltpu.VMEM_SHARED((V + WWIN,), jnp.int32),
        pltpu.VMEM_SHARED((NS, L), jnp.int32),
        pltpu.VMEM((SEG,), jnp.int32),
        pltpu.VMEM((WWIN,), jnp.int32),
        pltpu.VMEM((NS, L), jnp.int32),
        pltpu.VMEM((L,), jnp.int32),
        pltpu.VMEM((IBUF,), jnp.int32),
        pltpu.VMEM((NBUF, GB, D), jnp.float32),
        [pltpu.SemaphoreType.DMA] * NBUF,
        [pltpu.SemaphoreType.DMA] * NBUF,
    ],
    compiler_params=_params,
)
def _compact_gather(mask_hbm, states_hbm, out_hbm, smask, scnt, segbuf, wbuf,
                    cnts, cntrow, ibuf, rowbuf, gsems, wsems):
    cid = lax.axis_index("c")
    sid = lax.axis_index("s")
    wid = sid * NC + cid
    lane = lax.iota(jnp.int32, L)
    target = jnp.full((L,), wid * BPW, jnp.int32)

    with jax.named_scope("stage"):
        pltpu.sync_copy(mask_hbm.at[pl.ds(sid * SEG, SEG)], segbuf)

        def cbody(c, acc):
            for k in range(8):
                acc = acc + plsc.all_reduce_population_count(
                    segbuf[pl.ds(c * (8 * L) + k * L, L)] > 0)
            return acc

        segcnt = lax.fori_loop(0, SVR // 8, cbody, jnp.zeros((L,), jnp.int32))
        pltpu.sync_copy(segbuf, smask.at[pl.ds(sid * SEG, SEG)])
        cntrow[pl.ds(0, L)] = segcnt
        pltpu.sync_copy(cntrow, scnt.at[sid])
        plsc.subcore_barrier()

    with jax.named_scope("locate"):
        pltpu.sync_copy(scnt, cnts)
        zero = jnp.zeros((L,), jnp.int32)
        acc, startseg, accseg = zero, zero, zero
        for s in range(NS):
            cnt = cnts[s, pl.ds(0, L)]
            take = (acc + cnt) <= target
            startseg = startseg + jnp.where(take, 1, 0)
            accseg = jnp.where(take, acc + cnt, accseg)
            acc = acc + cnt
        sstar = jnp.max(startseg)
        pltpu.sync_copy(smask.at[pl.ds(sstar * SEG, SEG)], segbuf)

        def lbody(c, carry):
            acc2, startv, accsel = carry
            for k in range(8):
                cnt = plsc.all_reduce_population_count(
                    segbuf[pl.ds(c * (8 * L) + k * L, L)] > 0)
                take = (acc2 + cnt) <= (target - accseg)
                startv = startv + jnp.where(take, 1, 0)
                accsel = jnp.where(take, acc2 + cnt, accsel)
                acc2 = acc2 + cnt
            return acc2, startv, accsel

        _, startv, accsel = lax.fori_loop(0, SVR // 8, lbody,
                                          (zero, zero, zero))
        k0 = target - accseg - accsel
        v0 = sstar * SVR + jnp.max(startv)
        a = v0 * L

    with jax.named_scope("walk"):
        pltpu.sync_copy(smask.at[pl.ds(a, WWIN)], wbuf)
        m0 = wbuf[pl.ds(0, L)]
        ison0 = m0 > 0
        one0 = jnp.where(ison0, 1, 0)
        pref0 = plsc.cumsum(one0) - one0
        keep0 = ison0 & (pref0 >= k0)
        plsc.store_compressed(ibuf.at[pl.ds(0, L)],
                              jnp.full((L,), a, jnp.int32) + lane, mask=keep0)
        coll0 = jnp.sum(jnp.where(keep0, 1, 0))

        def wcond(carry):
            coll, vi = carry
            return (coll < BPW) & (vi < NWV)

        def wbody(carry):
            coll, vi = carry
            m = wbuf[pl.ds(vi * L, L)]
            ison = m > 0
            pos = jnp.full((L,), a + vi * L, jnp.int32) + lane
            plsc.store_compressed(ibuf.at[pl.ds(coll, L)], pos, mask=ison)
            return coll + jnp.sum(jnp.where(ison, 1, 0)), vi + 1

        lax.while_loop(wcond, wbody, (coll0, jnp.int32(1)))

        vmax = jnp.full((L,), V - 1, jnp.int32)
        vmin = jnp.zeros((L,), jnp.int32)
        for t in range(IBUF // L):
            ibuf[pl.ds(t * L, L)] = jnp.clip(ibuf[pl.ds(t * L, L)], vmin,
                                             vmax)

    with jax.named_scope("gather"):
        obase = wid * BPW
        gh = [None] * NGB
        wh = [None] * NGB
        for j in range(AHEAD):
            gh[j] = pltpu.async_copy(
                states_hbm.at[ibuf.at[pl.ds(j * GB, GB)]],
                rowbuf.at[j % NBUF], gsems[j % NBUF])
        for j in range(NGB):
            b = j % NBUF
            gh[j].wait()
            wh[j] = pltpu.async_copy(
                rowbuf.at[b], out_hbm.at[pl.ds(obase + j * GB, GB)],
                wsems[b])
            nj = j + AHEAD
            if nj < NGB:
                nb = nj % NBUF
                if nj - NBUF >= 0:
                    wh[nj - NBUF].wait()
                gh[nj] = pltpu.async_copy(
                    states_hbm.at[ibuf.at[pl.ds(nj * GB, GB)]],
                    rowbuf.at[nb], gsems[nb])
        for j in range(NGB - NBUF, NGB):
            wh[j].wait()


def kernel(inputs, states, importance_scores, active_mask):
    return _compact_gather(active_mask.astype(jnp.int32), states)
